# speedup vs baseline: 2.7398x; 2.7398x over previous
import jax
import jax.numpy as jnp
from jax.experimental import pallas as pl

_BLOCK_ROWS = 512


def _copy_block(w_ref, o_ref):
    o_ref[...] = w_ref[...]


def kernel(x, emb_weight):
    seq_len = x.shape[1]
    model_dim = emb_weight.shape[1]
    table = emb_weight[:seq_len]
    rows = table.shape[0]
    block = min(_BLOCK_ROWS, rows)
    grid = (pl.cdiv(rows, block),)
    return pl.pallas_call(
        _copy_block,
        grid=grid,
        in_specs=[pl.BlockSpec((block, model_dim), lambda i: (i, 0))],
        out_specs=pl.BlockSpec((block, model_dim), lambda i: (i, 0)),
        out_shape=jax.ShapeDtypeStruct((rows, model_dim), table.dtype),
    )(table)
